# baseline (speedup 1.0000x reference)
import functools

import jax
import jax.numpy as jnp
from jax import lax
from jax.experimental import pallas as pl
from jax.experimental.pallas import tpu as pltpu
from jax.experimental.pallas import tpu_sc as plsc

B = 16384
EMB = 32
H1 = 64
NF = 26

_NC, _NS = 2, 16
_NW = _NC * _NS
_BPW = B // _NW
_CH = 128
_NCH = _BPW // _CH


def _sc_gather(u2, g2, user_emb, game_emb):
    mesh = plsc.VectorSubcoreMesh(core_axis_name="c", subcore_axis_name="s")

    @functools.partial(
        pl.kernel,
        mesh=mesh,
        out_type=[
            jax.ShapeDtypeStruct((B, EMB), jnp.float32),
            jax.ShapeDtypeStruct((B, EMB), jnp.float32),
        ],
        scratch_types=[
            pltpu.VMEM((_NCH, _CH), jnp.int32),
            pltpu.VMEM((_NCH, _CH), jnp.int32),
            pltpu.VMEM((_BPW, EMB), jnp.float32),
            pltpu.VMEM((_BPW, EMB), jnp.float32),
            pltpu.SemaphoreType.DMA,
        ],
        compiler_params=pltpu.CompilerParams(use_tc_tiling_on_sc=False),
    )
    def gather(u_hbm, g_hbm, ue_tab, ge_tab, ue_out, ge_out,
               u_v, g_v, ur_v, gr_v, sem):
        wid = lax.axis_index("s") * _NC + lax.axis_index("c")
        pltpu.sync_copy(u_hbm.at[pl.ds(wid * _NCH, _NCH)], u_v)
        pltpu.sync_copy(g_hbm.at[pl.ds(wid * _NCH, _NCH)], g_v)
        copies = []
        for k in range(_NCH):
            copies.append(pltpu.async_copy(
                ue_tab.at[u_v.at[k]], ur_v.at[pl.ds(k * _CH, _CH)], sem))
            copies.append(pltpu.async_copy(
                ge_tab.at[g_v.at[k]], gr_v.at[pl.ds(k * _CH, _CH)], sem))
        for c in copies:
            c.wait()
        base = wid * _BPW
        pltpu.sync_copy(ur_v, ue_out.at[pl.ds(base, _BPW)])
        pltpu.sync_copy(gr_v, ge_out.at[pl.ds(base, _BPW)])

    return gather(u2, g2, user_emb, game_emb)


_R = 2048
_G = B // _R


def _mlp_body(ue_ref, ge_ref, f_ref, wft, bf_r, w1u, w1g, w1f, b1_r,
              w2_r, b2_r, out_ref):
    fe = jnp.dot(f_ref[...], wft[...],
                 preferred_element_type=jnp.float32) + bf_r[...]
    h = (jnp.dot(ue_ref[...], w1u[...], preferred_element_type=jnp.float32)
         + jnp.dot(ge_ref[...], w1g[...], preferred_element_type=jnp.float32)
         + jnp.dot(fe, w1f[...], preferred_element_type=jnp.float32)
         + b1_r[...])
    h = jnp.maximum(h, 0.0)
    out_ref[...] = jnp.dot(h, w2_r[...],
                           preferred_element_type=jnp.float32) + b2_r[...]


def _tc_mlp(ue, ge, f, wft, bf2, w1ut, w1gt, w1ft, b12, w2t, b22,
            interpret=False):
    row = lambda i: (i, 0)
    rep = lambda i: (0, 0)
    return pl.pallas_call(
        _mlp_body,
        grid=(_G,),
        in_specs=[
            pl.BlockSpec((_R, EMB), row),
            pl.BlockSpec((_R, EMB), row),
            pl.BlockSpec((_R, NF), row),
            pl.BlockSpec((NF, EMB), rep),
            pl.BlockSpec((1, EMB), rep),
            pl.BlockSpec((EMB, H1), rep),
            pl.BlockSpec((EMB, H1), rep),
            pl.BlockSpec((EMB, H1), rep),
            pl.BlockSpec((1, H1), rep),
            pl.BlockSpec((H1, 1), rep),
            pl.BlockSpec((1, 1), rep),
        ],
        out_specs=pl.BlockSpec((_R, 1), row),
        out_shape=jax.ShapeDtypeStruct((B, 1), jnp.float32),
        interpret=interpret,
    )(ue, ge, f, wft, bf2, w1ut, w1gt, w1ft, b12, w2t, b22)


def kernel(u, g, f, user_emb, game_emb, Wf, bf, W1, b1, W2, b2):
    u2 = u.reshape(_NW * _NCH, _CH)
    g2 = g.reshape(_NW * _NCH, _CH)
    ue, ge = _sc_gather(u2, g2, user_emb, game_emb)
    return _tc_mlp(
        ue, ge, f,
        Wf.T,
        bf.reshape(1, EMB),
        W1[:, :EMB].T,
        W1[:, EMB:2 * EMB].T,
        W1[:, 2 * EMB:].T,
        b1.reshape(1, H1),
        W2.T,
        b2.reshape(1, 1),
    )
